# baseline (speedup 1.0000x reference)
import functools

import jax
import jax.numpy as jnp
from jax import lax
from jax.experimental import pallas as pl
from jax.experimental.pallas import tpu as pltpu
from jax.experimental.pallas import tpu_sc as plsc

VOCAB = 100000
EMBED = 64
BATCH = 1024

_NUM_CORES = 2
_NUM_SUBCORES = 16
_NW = _NUM_CORES * _NUM_SUBCORES
_BPW = BATCH // _NW

_VT = 2048
_NSTEP = pl.cdiv(VOCAB, _VT)
_VLAST = VOCAB - (_NSTEP - 1) * _VT
_NBUF = 3
_NSTRIPE = 4
_ROWS = BATCH // _NSTRIPE


@functools.partial(
    pl.kernel,
    out_type=jax.ShapeDtypeStruct((BATCH, 2 * EMBED), jnp.float32),
    mesh=plsc.VectorSubcoreMesh(
        core_axis_name="c", subcore_axis_name="s",
        num_cores=_NUM_CORES, num_subcores=_NUM_SUBCORES),
    scratch_types=[
        pltpu.VMEM((_BPW,), jnp.int32),
        pltpu.VMEM((_BPW, 2 * EMBED), jnp.float32),
        pltpu.SemaphoreType.DMA,
    ],
)
def _sc_gather(table2_hbm, idx2_hbm, out_hbm, idx_v, rows_v, sem):
    wid = lax.axis_index("s") * _NUM_CORES + lax.axis_index("c")
    base = wid * _BPW
    pltpu.sync_copy(idx2_hbm.at[pl.ds(base, _BPW)], idx_v)
    pltpu.async_copy(table2_hbm.at[idx_v], rows_v, sem).wait()
    pltpu.sync_copy(rows_v, out_hbm.at[pl.ds(base, _BPW)])


def _fused_body(par_ref, hid2_ref, w_ref, o_ref, hidden_s):
    j = pl.program_id(0)

    @pl.when(j == 0)
    def _select():
        h2 = hid2_ref[...]
        hidden_s[...] = jnp.where(par_ref[...] == 1,
                                  h2[:, EMBED:], h2[:, :EMBED])

    o_ref[...] = lax.dot_general(
        hidden_s[...], w_ref[...],
        dimension_numbers=(((1,), (1,)), ((), ())),
        preferred_element_type=jnp.float32)


def _fused(idx, embed_table, expand_w):
    table2 = embed_table.reshape(VOCAB // 2, 2 * EMBED)
    idx2 = idx // 2
    par = (idx & 1).reshape(BATCH, 1)
    hidden2 = _sc_gather(table2, idx2)
    return pl.pallas_call(
        _fused_body,
        grid=(_NSTEP,),
        in_specs=[
            pl.BlockSpec((BATCH, 1), lambda j: (0, 0)),
            pl.BlockSpec((BATCH, 2 * EMBED), lambda j: (0, 0)),
            pl.BlockSpec((_VT, EMBED), lambda j: (j, 0)),
        ],
        out_specs=pl.BlockSpec((BATCH, _VT), lambda j: (0, j)),
        out_shape=jax.ShapeDtypeStruct((BATCH, VOCAB), jnp.float32),
        scratch_shapes=[
            pltpu.VMEM((BATCH, EMBED), jnp.float32),
        ],
    )(par, hidden2, expand_w)


def _mm_body(h_ref, w_ref, o_ref, scratch, last, sems):
    j = pl.program_id(0)
    buf = lax.rem(j, _NBUF)

    def _ring_copies(b, step):
        return [
            pltpu.make_async_copy(
                scratch.at[b, pl.ds(s * _ROWS, _ROWS), :],
                o_ref.at[pl.ds(s * _ROWS, _ROWS), pl.ds(step * _VT, _VT)],
                sems.at[b, s],
            )
            for s in range(_NSTRIPE)
        ]

    def _last_copies():
        return [
            pltpu.make_async_copy(
                last.at[pl.ds(s * _ROWS, _ROWS), :],
                o_ref.at[pl.ds(s * _ROWS, _ROWS),
                         pl.ds((_NSTEP - 1) * _VT, _VLAST)],
                sems.at[_NBUF, s],
            )
            for s in range(_NSTRIPE)
        ]

    @pl.when(j >= _NBUF)
    def _wait_ring():
        for cp in _ring_copies(buf, j - _NBUF):
            cp.wait()

    res = lax.dot_general(
        h_ref[...], w_ref[...],
        dimension_numbers=(((1,), (1,)), ((), ())),
        preferred_element_type=jnp.float32)

    @pl.when(j < _NSTEP - 1)
    def _push_ring():
        scratch[buf] = res
        for cp in _ring_copies(buf, j):
            cp.start()

    @pl.when(j == _NSTEP - 1)
    def _push_last_and_drain():
        last[...] = res[:, :_VLAST]
        for cp in _last_copies():
            cp.start()
        for d in (2, 1):
            step = _NSTEP - 1 - d
            for cp in _ring_copies(lax.rem(step, _NBUF), step):
                cp.wait()
        for cp in _last_copies():
            cp.wait()


def _project(hidden, expand_w):
    return pl.pallas_call(
        _mm_body,
        grid=(_NSTEP,),
        in_specs=[
            pl.BlockSpec((BATCH, EMBED), lambda j: (0, 0)),
            pl.BlockSpec((_VT, EMBED), lambda j: (j, 0)),
        ],
        out_specs=pl.BlockSpec(memory_space=pl.ANY),
        out_shape=jax.ShapeDtypeStruct((BATCH, VOCAB), jnp.float32),
        scratch_shapes=[
            pltpu.VMEM((_NBUF, BATCH, _VT), jnp.float32),
            pltpu.VMEM((BATCH, _VLAST), jnp.float32),
            pltpu.SemaphoreType.DMA((_NBUF + 1, _NSTRIPE)),
        ],
    )(hidden, expand_w)


@jax.jit
def kernel(input, embed_table, expand_w):
    idx = input.astype(jnp.int32)
    return _fused(idx, embed_table, expand_w)


# speedup vs baseline: 1.0406x; 1.0406x over previous
import functools

import jax
import jax.numpy as jnp
from jax import lax
from jax.experimental import pallas as pl
from jax.experimental.pallas import tpu as pltpu
from jax.experimental.pallas import tpu_sc as plsc

VOCAB = 100000
EMBED = 64
BATCH = 1024

_NUM_CORES = 2
_NUM_SUBCORES = 16
_NW = _NUM_CORES * _NUM_SUBCORES
_BPW = BATCH // _NW

_VT = 2048
_NSTEP = pl.cdiv(VOCAB, _VT)
_VLAST = VOCAB - (_NSTEP - 1) * _VT
_NBUF = 3
_NSTRIPE = 4
_ROWS = BATCH // _NSTRIPE


@functools.partial(
    pl.kernel,
    out_type=jax.ShapeDtypeStruct((BATCH, 2 * EMBED), jnp.float32),
    mesh=plsc.VectorSubcoreMesh(
        core_axis_name="c", subcore_axis_name="s",
        num_cores=_NUM_CORES, num_subcores=_NUM_SUBCORES),
    scratch_types=[
        pltpu.VMEM((_BPW,), jnp.int32),
        pltpu.VMEM((_BPW, 2 * EMBED), jnp.float32),
        pltpu.SemaphoreType.DMA,
    ],
)
def _sc_gather(table2_hbm, idx2_hbm, out_hbm, idx_v, rows_v, sem):
    wid = lax.axis_index("s") * _NUM_CORES + lax.axis_index("c")
    base = wid * _BPW
    pltpu.sync_copy(idx2_hbm.at[pl.ds(base, _BPW)], idx_v)
    pltpu.async_copy(table2_hbm.at[idx_v], rows_v, sem).wait()
    pltpu.sync_copy(rows_v, out_hbm.at[pl.ds(base, _BPW)])


_CHUNK = 32


def _fused_body(idx_ref, mod_ref, table_ref, w_ref, o_ref,
                blocks_s, hidden_s, sem):
    j = pl.program_id(0)

    def _cp(i, b):
        return pltpu.make_async_copy(
            table_ref.at[pl.ds(b, 8), :], blocks_s.at[i], sem)

    @pl.when(j == 0)
    def _gather():
        def issue(i, c):
            base = i * 8
            for u in range(8):
                r = idx_ref[base + u]
                _cp(base + u, (r // 8) * 8).start()
            return c
        lax.fori_loop(0, BATCH // 8, issue, 0)

        def drain(i, c):
            _cp(0, 0).wait()
            return c
        lax.fori_loop(0, BATCH, drain, 0)

        def select(c, carry):
            v = blocks_s[pl.ds(c * _CHUNK, _CHUNK)]
            m = mod_ref[pl.ds(c * _CHUNK, _CHUNK)]
            acc = v[:, 0, :]
            for r in range(1, 8):
                acc = jnp.where(m == r, v[:, r, :], acc)
            hidden_s[pl.ds(c * _CHUNK, _CHUNK), :] = acc
            return carry
        lax.fori_loop(0, BATCH // _CHUNK, select, 0)

    o_ref[...] = lax.dot_general(
        hidden_s[...], w_ref[...],
        dimension_numbers=(((1,), (1,)), ((), ())),
        preferred_element_type=jnp.float32)


def _fused(idx, embed_table, expand_w):
    mod = (idx % 8).reshape(BATCH, 1)
    return pl.pallas_call(
        _fused_body,
        grid=(_NSTEP,),
        in_specs=[
            pl.BlockSpec(memory_space=pltpu.SMEM),
            pl.BlockSpec((BATCH, 1), lambda j: (0, 0)),
            pl.BlockSpec(memory_space=pl.ANY),
            pl.BlockSpec((_VT, EMBED), lambda j: (j, 0)),
        ],
        out_specs=pl.BlockSpec((BATCH, _VT), lambda j: (0, j)),
        out_shape=jax.ShapeDtypeStruct((BATCH, VOCAB), jnp.float32),
        scratch_shapes=[
            pltpu.VMEM((BATCH, 8, EMBED), jnp.float32),
            pltpu.VMEM((BATCH, EMBED), jnp.float32),
            pltpu.SemaphoreType.DMA,
        ],
    )(idx, mod, embed_table, expand_w)


def _mm_body(h_ref, w_ref, o_ref, scratch, last, sems):
    j = pl.program_id(0)
    buf = lax.rem(j, _NBUF)

    def _ring_copies(b, step):
        return [
            pltpu.make_async_copy(
                scratch.at[b, pl.ds(s * _ROWS, _ROWS), :],
                o_ref.at[pl.ds(s * _ROWS, _ROWS), pl.ds(step * _VT, _VT)],
                sems.at[b, s],
            )
            for s in range(_NSTRIPE)
        ]

    def _last_copies():
        return [
            pltpu.make_async_copy(
                last.at[pl.ds(s * _ROWS, _ROWS), :],
                o_ref.at[pl.ds(s * _ROWS, _ROWS),
                         pl.ds((_NSTEP - 1) * _VT, _VLAST)],
                sems.at[_NBUF, s],
            )
            for s in range(_NSTRIPE)
        ]

    @pl.when(j >= _NBUF)
    def _wait_ring():
        for cp in _ring_copies(buf, j - _NBUF):
            cp.wait()

    res = lax.dot_general(
        h_ref[...], w_ref[...],
        dimension_numbers=(((1,), (1,)), ((), ())),
        preferred_element_type=jnp.float32)

    @pl.when(j < _NSTEP - 1)
    def _push_ring():
        scratch[buf] = res
        for cp in _ring_copies(buf, j):
            cp.start()

    @pl.when(j == _NSTEP - 1)
    def _push_last_and_drain():
        last[...] = res[:, :_VLAST]
        for cp in _last_copies():
            cp.start()
        for d in (2, 1):
            step = _NSTEP - 1 - d
            for cp in _ring_copies(lax.rem(step, _NBUF), step):
                cp.wait()
        for cp in _last_copies():
            cp.wait()


def _project(hidden, expand_w):
    return pl.pallas_call(
        _mm_body,
        grid=(_NSTEP,),
        in_specs=[
            pl.BlockSpec((BATCH, EMBED), lambda j: (0, 0)),
            pl.BlockSpec((_VT, EMBED), lambda j: (j, 0)),
        ],
        out_specs=pl.BlockSpec(memory_space=pl.ANY),
        out_shape=jax.ShapeDtypeStruct((BATCH, VOCAB), jnp.float32),
        scratch_shapes=[
            pltpu.VMEM((_NBUF, BATCH, _VT), jnp.float32),
            pltpu.VMEM((BATCH, _VLAST), jnp.float32),
            pltpu.SemaphoreType.DMA((_NBUF + 1, _NSTRIPE)),
        ],
    )(hidden, expand_w)


@jax.jit
def kernel(input, embed_table, expand_w):
    idx = input.astype(jnp.int32)
    return _fused(idx, embed_table, expand_w)


# speedup vs baseline: 1.0815x; 1.0393x over previous
import functools

import jax
import jax.numpy as jnp
from jax import lax
from jax.experimental import pallas as pl
from jax.experimental.pallas import tpu as pltpu
from jax.experimental.pallas import tpu_sc as plsc

VOCAB = 100000
EMBED = 64
BATCH = 1024

_NUM_CORES = 2
_NUM_SUBCORES = 16
_NW = _NUM_CORES * _NUM_SUBCORES
_BPW = BATCH // _NW

_VT = 2048
_NSTEP = pl.cdiv(VOCAB, _VT)
_VLAST = VOCAB - (_NSTEP - 1) * _VT
_NBUF = 3
_NSTRIPE = 4
_ROWS = BATCH // _NSTRIPE


@functools.partial(
    pl.kernel,
    out_type=jax.ShapeDtypeStruct((BATCH, 2 * EMBED), jnp.float32),
    mesh=plsc.VectorSubcoreMesh(
        core_axis_name="c", subcore_axis_name="s",
        num_cores=_NUM_CORES, num_subcores=_NUM_SUBCORES),
    scratch_types=[
        pltpu.VMEM((_BPW,), jnp.int32),
        pltpu.VMEM((_BPW, 2 * EMBED), jnp.float32),
        pltpu.SemaphoreType.DMA,
    ],
)
def _sc_gather(table2_hbm, idx2_hbm, out_hbm, idx_v, rows_v, sem):
    wid = lax.axis_index("s") * _NUM_CORES + lax.axis_index("c")
    base = wid * _BPW
    pltpu.sync_copy(idx2_hbm.at[pl.ds(base, _BPW)], idx_v)
    pltpu.async_copy(table2_hbm.at[idx_v], rows_v, sem).wait()
    pltpu.sync_copy(rows_v, out_hbm.at[pl.ds(base, _BPW)])


_CHUNK = 32


def _fused_body(idx_ref, mod_ref, table_ref, w_ref, o_ref,
                blocks_s, hidden_s, sem):
    j = pl.program_id(0)

    def _cp(i, b):
        return pltpu.make_async_copy(
            table_ref.at[pl.ds(b, 8), :], blocks_s.at[i], sem)

    @pl.when(j == 0)
    def _gather():
        return
        def issue(i, c):
            base = i * 8
            for u in range(8):
                r = idx_ref[base + u]
                _cp(base + u, (r // 8) * 8).start()
            return c
        lax.fori_loop(0, BATCH // 8, issue, 0)

        def drain(i, c):
            _cp(0, 0).wait()
            return c
        lax.fori_loop(0, BATCH, drain, 0)

        def select(c, carry):
            v = blocks_s[pl.ds(c * _CHUNK, _CHUNK)]
            m = mod_ref[pl.ds(c * _CHUNK, _CHUNK)]
            acc = v[:, 0, :]
            for r in range(1, 8):
                acc = jnp.where(m == r, v[:, r, :], acc)
            hidden_s[pl.ds(c * _CHUNK, _CHUNK), :] = acc
            return carry
        lax.fori_loop(0, BATCH // _CHUNK, select, 0)

    o_ref[...] = lax.dot_general(
        hidden_s[...], w_ref[...],
        dimension_numbers=(((1,), (1,)), ((), ())),
        preferred_element_type=jnp.float32)


def _fused(idx, embed_table, expand_w):
    mod = (idx % 8).reshape(BATCH, 1)
    return pl.pallas_call(
        _fused_body,
        grid=(_NSTEP,),
        in_specs=[
            pl.BlockSpec(memory_space=pltpu.SMEM),
            pl.BlockSpec((BATCH, 1), lambda j: (0, 0)),
            pl.BlockSpec(memory_space=pl.ANY),
            pl.BlockSpec((_VT, EMBED), lambda j: (j, 0)),
        ],
        out_specs=pl.BlockSpec((BATCH, _VT), lambda j: (0, j)),
        out_shape=jax.ShapeDtypeStruct((BATCH, VOCAB), jnp.float32),
        scratch_shapes=[
            pltpu.VMEM((BATCH, 8, EMBED), jnp.float32),
            pltpu.VMEM((BATCH, EMBED), jnp.float32),
            pltpu.SemaphoreType.DMA,
        ],
    )(idx, mod, embed_table, expand_w)


def _mm_body(h_ref, w_ref, o_ref, scratch, last, sems):
    j = pl.program_id(0)
    buf = lax.rem(j, _NBUF)

    def _ring_copies(b, step):
        return [
            pltpu.make_async_copy(
                scratch.at[b, pl.ds(s * _ROWS, _ROWS), :],
                o_ref.at[pl.ds(s * _ROWS, _ROWS), pl.ds(step * _VT, _VT)],
                sems.at[b, s],
            )
            for s in range(_NSTRIPE)
        ]

    def _last_copies():
        return [
            pltpu.make_async_copy(
                last.at[pl.ds(s * _ROWS, _ROWS), :],
                o_ref.at[pl.ds(s * _ROWS, _ROWS),
                         pl.ds((_NSTEP - 1) * _VT, _VLAST)],
                sems.at[_NBUF, s],
            )
            for s in range(_NSTRIPE)
        ]

    @pl.when(j >= _NBUF)
    def _wait_ring():
        for cp in _ring_copies(buf, j - _NBUF):
            cp.wait()

    res = lax.dot_general(
        h_ref[...], w_ref[...],
        dimension_numbers=(((1,), (1,)), ((), ())),
        preferred_element_type=jnp.float32)

    @pl.when(j < _NSTEP - 1)
    def _push_ring():
        scratch[buf] = res
        for cp in _ring_copies(buf, j):
            cp.start()

    @pl.when(j == _NSTEP - 1)
    def _push_last_and_drain():
        last[...] = res[:, :_VLAST]
        for cp in _last_copies():
            cp.start()
        for d in (2, 1):
            step = _NSTEP - 1 - d
            for cp in _ring_copies(lax.rem(step, _NBUF), step):
                cp.wait()
        for cp in _last_copies():
            cp.wait()


def _project(hidden, expand_w):
    return pl.pallas_call(
        _mm_body,
        grid=(_NSTEP,),
        in_specs=[
            pl.BlockSpec((BATCH, EMBED), lambda j: (0, 0)),
            pl.BlockSpec((_VT, EMBED), lambda j: (j, 0)),
        ],
        out_specs=pl.BlockSpec(memory_space=pl.ANY),
        out_shape=jax.ShapeDtypeStruct((BATCH, VOCAB), jnp.float32),
        scratch_shapes=[
            pltpu.VMEM((_NBUF, BATCH, _VT), jnp.float32),
            pltpu.VMEM((BATCH, _VLAST), jnp.float32),
            pltpu.SemaphoreType.DMA((_NBUF + 1, _NSTRIPE)),
        ],
    )(hidden, expand_w)


@jax.jit
def kernel(input, embed_table, expand_w):
    idx = input.astype(jnp.int32)
    return _fused(idx, embed_table, expand_w)


# speedup vs baseline: 3.4698x; 3.2082x over previous
import functools

import jax
import jax.numpy as jnp
from jax import lax
from jax.experimental import pallas as pl
from jax.experimental.pallas import tpu as pltpu
from jax.experimental.pallas import tpu_sc as plsc

VOCAB = 100000
EMBED = 64
BATCH = 1024

_NUM_CORES = 2
_NUM_SUBCORES = 16
_NW = _NUM_CORES * _NUM_SUBCORES
_BPW = BATCH // _NW

_VT = 2048
_NSTEP = pl.cdiv(VOCAB, _VT)
_VLAST = VOCAB - (_NSTEP - 1) * _VT
_NBUF = 3
_NSTRIPE = 4
_ROWS = BATCH // _NSTRIPE


@functools.partial(
    pl.kernel,
    out_type=jax.ShapeDtypeStruct((BATCH, 2 * EMBED), jnp.float32),
    mesh=plsc.VectorSubcoreMesh(
        core_axis_name="c", subcore_axis_name="s",
        num_cores=_NUM_CORES, num_subcores=_NUM_SUBCORES),
    scratch_types=[
        pltpu.VMEM((_BPW,), jnp.int32),
        pltpu.VMEM((_BPW, 2 * EMBED), jnp.float32),
        pltpu.SemaphoreType.DMA,
    ],
)
def _sc_gather(table2_hbm, idx2_hbm, out_hbm, idx_v, rows_v, sem):
    wid = lax.axis_index("s") * _NUM_CORES + lax.axis_index("c")
    base = wid * _BPW
    pltpu.sync_copy(idx2_hbm.at[pl.ds(base, _BPW)], idx_v)
    pltpu.async_copy(table2_hbm.at[idx_v], rows_v, sem).wait()
    pltpu.sync_copy(rows_v, out_hbm.at[pl.ds(base, _BPW)])


_CHUNK = 32


def _fused_body(idx_ref, mod_ref, tableT_ref, wT_ref, oT_ref,
                blocks_s, hidden_s, sem):
    j = pl.program_id(0)

    def _cp(i, b):
        return pltpu.make_async_copy(
            tableT_ref.at[:, pl.ds(b, 128)], blocks_s.at[i], sem)

    @pl.when(j == 0)
    def _gather():
        def issue(i, c):
            base = i * 8
            for u in range(8):
                col = idx_ref[base + u]
                b = pl.multiple_of((col // 128) * 128, 128)
                _cp(base + u, b).start()
            return c
        lax.fori_loop(0, BATCH // 8, issue, 0)

        def drain(i, c):
            _cp(0, 0).wait()
            return c
        lax.fori_loop(0, BATCH, drain, 0)

        def select(c, carry):
            v = blocks_s[pl.ds(c * 8, 8)]
            m = mod_ref[pl.ds(c * 8, 8)]
            hit = m == lax.broadcasted_iota(jnp.int32, (8, 128), 1)
            picked = jnp.where(hit[:, None, :], v, 0.0)
            hidden_s[pl.ds(c * 8, 8), :] = jnp.sum(picked, axis=2)
            return carry
        lax.fori_loop(0, BATCH // 8, select, 0)

    oT_ref[...] = lax.dot_general(
        wT_ref[...], hidden_s[...],
        dimension_numbers=(((0,), (1,)), ((), ())),
        preferred_element_type=jnp.float32)


def _fused(idx, embed_table, expand_w):
    mod = (idx % 128).reshape(BATCH, 1)
    logitsT = pl.pallas_call(
        _fused_body,
        grid=(_NSTEP,),
        in_specs=[
            pl.BlockSpec(memory_space=pltpu.SMEM),
            pl.BlockSpec((BATCH, 1), lambda j: (0, 0)),
            pl.BlockSpec(memory_space=pl.ANY),
            pl.BlockSpec((EMBED, _VT), lambda j: (0, j)),
        ],
        out_specs=pl.BlockSpec((_VT, BATCH), lambda j: (j, 0)),
        out_shape=jax.ShapeDtypeStruct((VOCAB, BATCH), jnp.float32),
        scratch_shapes=[
            pltpu.VMEM((BATCH, EMBED, 128), jnp.float32),
            pltpu.VMEM((BATCH, EMBED), jnp.float32),
            pltpu.SemaphoreType.DMA,
        ],
    )(idx, mod, embed_table.T, expand_w.T)
    return logitsT.T


def _mm_body(h_ref, w_ref, o_ref, scratch, last, sems):
    j = pl.program_id(0)
    buf = lax.rem(j, _NBUF)

    def _ring_copies(b, step):
        return [
            pltpu.make_async_copy(
                scratch.at[b, pl.ds(s * _ROWS, _ROWS), :],
                o_ref.at[pl.ds(s * _ROWS, _ROWS), pl.ds(step * _VT, _VT)],
                sems.at[b, s],
            )
            for s in range(_NSTRIPE)
        ]

    def _last_copies():
        return [
            pltpu.make_async_copy(
                last.at[pl.ds(s * _ROWS, _ROWS), :],
                o_ref.at[pl.ds(s * _ROWS, _ROWS),
                         pl.ds((_NSTEP - 1) * _VT, _VLAST)],
                sems.at[_NBUF, s],
            )
            for s in range(_NSTRIPE)
        ]

    @pl.when(j >= _NBUF)
    def _wait_ring():
        for cp in _ring_copies(buf, j - _NBUF):
            cp.wait()

    res = lax.dot_general(
        h_ref[...], w_ref[...],
        dimension_numbers=(((1,), (1,)), ((), ())),
        preferred_element_type=jnp.float32)

    @pl.when(j < _NSTEP - 1)
    def _push_ring():
        scratch[buf] = res
        for cp in _ring_copies(buf, j):
            cp.start()

    @pl.when(j == _NSTEP - 1)
    def _push_last_and_drain():
        last[...] = res[:, :_VLAST]
        for cp in _last_copies():
            cp.start()
        for d in (2, 1):
            step = _NSTEP - 1 - d
            for cp in _ring_copies(lax.rem(step, _NBUF), step):
                cp.wait()
        for cp in _last_copies():
            cp.wait()


def _project(hidden, expand_w):
    return pl.pallas_call(
        _mm_body,
        grid=(_NSTEP,),
        in_specs=[
            pl.BlockSpec((BATCH, EMBED), lambda j: (0, 0)),
            pl.BlockSpec((_VT, EMBED), lambda j: (j, 0)),
        ],
        out_specs=pl.BlockSpec(memory_space=pl.ANY),
        out_shape=jax.ShapeDtypeStruct((BATCH, VOCAB), jnp.float32),
        scratch_shapes=[
            pltpu.VMEM((_NBUF, BATCH, _VT), jnp.float32),
            pltpu.VMEM((BATCH, _VLAST), jnp.float32),
            pltpu.SemaphoreType.DMA((_NBUF + 1, _NSTRIPE)),
        ],
    )(hidden, expand_w)


@jax.jit
def kernel(input, embed_table, expand_w):
    idx = input.astype(jnp.int32)
    return _fused(idx, embed_table, expand_w)
